# baseline (speedup 1.0000x reference)
import functools

import jax
import jax.numpy as jnp
from jax import lax
from jax.experimental import pallas as pl
from jax.experimental.pallas import tpu as pltpu
from jax.experimental.pallas import tpu_sc as plsc

NUM_LODS = 10
TEX_H = 512
TEX_W = 512
NUM_CHANNELS = 11

NC = 2
NS = 16
NW = NC * NS
L = 16

CHUNK = 2048
SUB = 128
NSUB = CHUNK // SUB
DPAD = 16


def _compute_indices(slab_v, idx_v, iota):
    def jloop(j, c):
        for l in range(SUB // L):
            off = j * SUB + l * L
            p = (off + iota) * 3
            ys = plsc.load_gather(slab_v, [p])
            xs = plsc.load_gather(slab_v, [p + 1])
            lods = plsc.load_gather(slab_v, [p + 2])
            sy = lax.shift_right_logical(ys, lods)
            sx = lax.shift_right_logical(xs, lods)
            t = (1 << 20) - lax.shift_right_logical(
                jnp.full((L,), 1 << 20, jnp.int32), 2 * lods
            )
            base_row = t * jnp.int32(-1431655765)
            idx = base_row + lax.shift_left(sy, 9 - lods) + sx
            idx_v[j, l * L:(l + 1) * L] = idx
        return c

    lax.fori_loop(0, NSUB, jloop, 0)


def _tex_kernel_body(
    table_hbm, bi_hbm, out_hbm,
    slab0, slab1, idx0, idx1, rows0, rows1,
    sa0, sa1, sc0, sc1, sd0, sd1,
):
    wid = lax.axis_index("s") * NC + lax.axis_index("c")
    batch = out_hbm.shape[0]
    bpw = batch // NW
    nchunk = bpw // CHUNK
    iota = lax.iota(jnp.int32, L)

    slabs = [slab0, slab1]
    idxs = [idx0, idx1]
    rows = [rows0, rows1]
    sas = [sa0, sa1]
    scs = [sc0, sc1]
    sds = [sd0, sd1]

    def fire_a(k):
        base = wid * bpw + k * CHUNK
        return pltpu.async_copy(
            bi_hbm.at[pl.ds(base * 3, CHUNK * 3)], slabs[k % 2], sas[k % 2]
        )

    def fire_c(k):
        b = k % 2
        return [
            pltpu.async_copy(
                table_hbm.at[idxs[b].at[j]],
                rows[b].at[pl.ds(j * SUB, SUB)],
                scs[b],
            )
            for j in range(NSUB)
        ]

    def fire_d(k):
        base = wid * bpw + k * CHUNK
        return pltpu.async_copy(
            rows[k % 2], out_hbm.at[pl.ds(base, CHUNK)], sds[k % 2]
        )

    a_pend = {0: fire_a(0), 1: fire_a(1)}
    c_pend = {}
    d_pend = {}
    for k in range(nchunk):
        a_pend.pop(k).wait()
        _compute_indices(slabs[k % 2], idxs[k % 2], iota)
        if k + 2 < nchunk:
            a_pend[k + 2] = fire_a(k + 2)
        if k - 1 in c_pend:
            for cp in c_pend.pop(k - 1):
                cp.wait()
            d_pend[k - 1] = fire_d(k - 1)
        if k - 2 in d_pend:
            d_pend.pop(k - 2).wait()
        c_pend[k] = fire_c(k)
    for cp in c_pend.pop(nchunk - 1):
        cp.wait()
    d_pend[nchunk - 1] = fire_d(nchunk - 1)
    for k in sorted(d_pend):
        d_pend.pop(k).wait()


def _make_tex_gather(batch):
    mesh = plsc.VectorSubcoreMesh(
        core_axis_name="c", subcore_axis_name="s", num_cores=NC, num_subcores=NS
    )
    return functools.partial(
        pl.kernel,
        out_type=jax.ShapeDtypeStruct((batch, DPAD), jnp.float32),
        mesh=mesh,
        scratch_types=[
            pltpu.VMEM((CHUNK * 3,), jnp.int32),
            pltpu.VMEM((CHUNK * 3,), jnp.int32),
            pltpu.VMEM((NSUB, SUB), jnp.int32),
            pltpu.VMEM((NSUB, SUB), jnp.int32),
            pltpu.VMEM((CHUNK, DPAD), jnp.float32),
            pltpu.VMEM((CHUNK, DPAD), jnp.float32),
            pltpu.SemaphoreType.DMA,
            pltpu.SemaphoreType.DMA,
            pltpu.SemaphoreType.DMA,
            pltpu.SemaphoreType.DMA,
            pltpu.SemaphoreType.DMA,
            pltpu.SemaphoreType.DMA,
        ],
        compiler_params=pltpu.CompilerParams(
            needs_layout_passes=False, use_tc_tiling_on_sc=False
        ),
    )(_tex_kernel_body)


def kernel(lod_cache, batch_index):
    batch = batch_index.shape[0]
    parts = [
        lax.slice(
            lod_cache,
            (l, 0, 0, 0),
            (l + 1, TEX_H >> l, TEX_W >> l, NUM_CHANNELS),
        ).reshape(-1, NUM_CHANNELS)
        for l in range(NUM_LODS)
    ]
    table = jnp.concatenate(parts, axis=0)
    nrows = table.shape[0]
    rpad = (-nrows) % 8
    table = jnp.pad(table, ((0, rpad), (0, DPAD - NUM_CHANNELS)))
    bi = batch_index.astype(jnp.int32).reshape(-1)
    out = _make_tex_gather(batch)(table, bi)
    return out[:, :NUM_CHANNELS]


# speedup vs baseline: 1.0269x; 1.0269x over previous
import functools

import jax
import jax.numpy as jnp
from jax import lax
from jax.experimental import pallas as pl
from jax.experimental.pallas import tpu as pltpu
from jax.experimental.pallas import tpu_sc as plsc

NUM_LODS = 10
TEX_H = 512
TEX_W = 512
NUM_CHANNELS = 11

NC = 2
NS = 16
NW = NC * NS
L = 16

CHUNK = 2048
SUB = 128
NSUB = CHUNK // SUB
DPAD = 11


def _compute_indices(slab_v, idx_v, iota):
    def jloop(j, c):
        for l in range(SUB // L):
            off = j * SUB + l * L
            p = (off + iota) * 3
            ys = plsc.load_gather(slab_v, [p])
            xs = plsc.load_gather(slab_v, [p + 1])
            lods = plsc.load_gather(slab_v, [p + 2])
            sy = lax.shift_right_logical(ys, lods)
            sx = lax.shift_right_logical(xs, lods)
            t = (1 << 20) - lax.shift_right_logical(
                jnp.full((L,), 1 << 20, jnp.int32), 2 * lods
            )
            base_row = t * jnp.int32(-1431655765)
            idx = base_row + lax.shift_left(sy, 9 - lods) + sx
            idx_v[j, l * L:(l + 1) * L] = idx
        return c

    lax.fori_loop(0, NSUB, jloop, 0)


def _tex_kernel_body(
    table_hbm, bi_hbm, out_hbm,
    slab0, slab1, idx0, idx1, rows0, rows1,
    sa0, sa1, sc0, sc1, sd0, sd1,
):
    wid = lax.axis_index("s") * NC + lax.axis_index("c")
    batch = out_hbm.shape[0]
    bpw = batch // NW
    nchunk = bpw // CHUNK
    iota = lax.iota(jnp.int32, L)

    slabs = [slab0, slab1]
    idxs = [idx0, idx1]
    rows = [rows0, rows1]
    sas = [sa0, sa1]
    scs = [sc0, sc1]
    sds = [sd0, sd1]

    def fire_a(k):
        base = wid * bpw + k * CHUNK
        return pltpu.async_copy(
            bi_hbm.at[pl.ds(base * 3, CHUNK * 3)], slabs[k % 2], sas[k % 2]
        )

    def fire_c(k):
        b = k % 2
        return [
            pltpu.async_copy(
                table_hbm.at[idxs[b].at[j]],
                rows[b].at[pl.ds(j * SUB, SUB)],
                scs[b],
            )
            for j in range(NSUB)
        ]

    def fire_d(k):
        base = wid * bpw + k * CHUNK
        return pltpu.async_copy(
            rows[k % 2], out_hbm.at[pl.ds(base, CHUNK)], sds[k % 2]
        )

    a_pend = {0: fire_a(0), 1: fire_a(1)}
    c_pend = {}
    d_pend = {}
    for k in range(nchunk):
        a_pend.pop(k).wait()
        _compute_indices(slabs[k % 2], idxs[k % 2], iota)
        if k + 2 < nchunk:
            a_pend[k + 2] = fire_a(k + 2)
        if k - 1 in c_pend:
            for cp in c_pend.pop(k - 1):
                cp.wait()
            d_pend[k - 1] = fire_d(k - 1)
        if k - 2 in d_pend:
            d_pend.pop(k - 2).wait()
        c_pend[k] = fire_c(k)
    for cp in c_pend.pop(nchunk - 1):
        cp.wait()
    d_pend[nchunk - 1] = fire_d(nchunk - 1)
    for k in sorted(d_pend):
        d_pend.pop(k).wait()


def _make_tex_gather(batch):
    mesh = plsc.VectorSubcoreMesh(
        core_axis_name="c", subcore_axis_name="s", num_cores=NC, num_subcores=NS
    )
    return functools.partial(
        pl.kernel,
        out_type=jax.ShapeDtypeStruct((batch, DPAD), jnp.float32),
        mesh=mesh,
        scratch_types=[
            pltpu.VMEM((CHUNK * 3,), jnp.int32),
            pltpu.VMEM((CHUNK * 3,), jnp.int32),
            pltpu.VMEM((NSUB, SUB), jnp.int32),
            pltpu.VMEM((NSUB, SUB), jnp.int32),
            pltpu.VMEM((CHUNK, DPAD), jnp.float32),
            pltpu.VMEM((CHUNK, DPAD), jnp.float32),
            pltpu.SemaphoreType.DMA,
            pltpu.SemaphoreType.DMA,
            pltpu.SemaphoreType.DMA,
            pltpu.SemaphoreType.DMA,
            pltpu.SemaphoreType.DMA,
            pltpu.SemaphoreType.DMA,
        ],
        compiler_params=pltpu.CompilerParams(
            needs_layout_passes=False, use_tc_tiling_on_sc=False
        ),
    )(_tex_kernel_body)


def kernel(lod_cache, batch_index):
    batch = batch_index.shape[0]
    parts = [
        lax.slice(
            lod_cache,
            (l, 0, 0, 0),
            (l + 1, TEX_H >> l, TEX_W >> l, NUM_CHANNELS),
        ).reshape(-1, NUM_CHANNELS)
        for l in range(NUM_LODS)
    ]
    table = jnp.concatenate(parts, axis=0)
    nrows = table.shape[0]
    rpad = (-nrows) % 8
    table = jnp.pad(table, ((0, rpad), (0, 0)))
    bi = batch_index.astype(jnp.int32).reshape(-1)
    return _make_tex_gather(batch)(table, bi)
